# Initial kernel scaffold; baseline (speedup 1.0000x reference)
import jax
import jax.numpy as jnp
from jax.experimental import pallas as pl


def kernel(inputs, seq_vectors):
    raise NotImplementedError("write your pallas kernel here")


# speedup vs baseline: 3.5956x; 3.5956x over previous
import dataclasses
import functools

import jax
import jax.numpy as jnp
from jax import lax
from jax.experimental import pallas as pl
from jax.experimental.pallas import tpu as pltpu
from jax.experimental.pallas import tpu_sc as plsc

L = 16
CHUNK = 6400


def _lookup_sc(idx_2d, table, n_chunks, F):
    mesh = plsc.VectorSubcoreMesh(core_axis_name="c", subcore_axis_name="s")

    cp = pltpu.CompilerParams()
    if "needs_layout_passes" in pltpu.CompilerParams.__dataclass_fields__:
        cp = dataclasses.replace(cp, needs_layout_passes=False)

    @functools.partial(
        pl.kernel,
        out_type=jax.ShapeDtypeStruct((n_chunks, CHUNK * F), jnp.float32),
        mesh=mesh,
        scratch_types=[pltpu.VMEM(table.shape, jnp.float32)],
        compiler_params=cp,
    )
    def run(table_hbm, idx_hbm, out_hbm, table_v):
        pltpu.sync_copy(table_hbm, table_v)

        def body(idx_v, out_v):
            lane = lax.iota(jnp.int32, L)
            zeros = jnp.zeros((L,), jnp.int32)

            @pl.loop(0, CHUNK // L)
            def _(g):
                iv = idx_v[0, pl.ds(g * L, L)]
                base = g * (L * F)
                for f in range(F):
                    vals = plsc.load_gather(
                        table_v, [iv, jnp.full((L,), f, jnp.int32)])
                    pos = base + lane * F + f
                    plsc.store_scatter(out_v, [zeros, pos], vals)

        pltpu.emit_pipeline(
            body,
            grid=(n_chunks,),
            in_specs=[pl.BlockSpec((1, CHUNK), lambda i: (i, 0))],
            out_specs=[pl.BlockSpec((1, CHUNK * F), lambda i: (i, 0))],
            core_axis_name=("c", "s"),
            dimension_semantics=(pltpu.PARALLEL,),
        )(idx_hbm, out_hbm)

    return run(table, idx_2d)


def kernel(inputs, seq_vectors):
    B, S = inputs.shape
    V, F = seq_vectors.shape
    N = B * S
    n_chunks = N // CHUNK
    idx_2d = inputs.reshape(n_chunks, CHUNK)
    out = _lookup_sc(idx_2d, seq_vectors, n_chunks, F)
    return out.reshape(B, S, F)


# speedup vs baseline: 29.3690x; 8.1679x over previous
import dataclasses
import functools

import jax
import jax.numpy as jnp
from jax.experimental import pallas as pl
from jax.experimental.pallas import tpu as pltpu
from jax.experimental.pallas import tpu_sc as plsc

L = 16
SB = 8
BB = 512


def _lookup_sc(idx_t, table):
    S, B = idx_t.shape
    V, F = table.shape
    mesh = plsc.VectorSubcoreMesh(core_axis_name="c", subcore_axis_name="s")

    cp = pltpu.CompilerParams()
    if "needs_layout_passes" in pltpu.CompilerParams.__dataclass_fields__:
        cp = dataclasses.replace(cp, needs_layout_passes=False)

    @functools.partial(
        pl.kernel,
        out_type=jax.ShapeDtypeStruct((F, S, B), jnp.float32),
        mesh=mesh,
        scratch_types=[pltpu.VMEM((V, F), jnp.float32)],
        compiler_params=cp,
    )
    def run(table_hbm, idx_hbm, out_hbm, table_v):
        pltpu.sync_copy(table_hbm, table_v)

        def body(idx_v, out_v):
            @pl.loop(0, SB)
            def _(s):
                @pl.loop(0, BB, step=L)
                def _(b):
                    iv = idx_v[s, pl.ds(b, L)]
                    for f in range(F):
                        vals = plsc.load_gather(
                            table_v, [iv, jnp.full((L,), f, jnp.int32)])
                        out_v[f, s, pl.ds(b, L)] = vals

        pltpu.emit_pipeline(
            body,
            grid=(S // SB, B // BB),
            in_specs=[pl.BlockSpec((SB, BB), lambda i, j: (i, j))],
            out_specs=[pl.BlockSpec((F, SB, BB), lambda i, j: (0, i, j))],
            core_axis_name=("c", "s"),
            dimension_semantics=(pltpu.PARALLEL, pltpu.PARALLEL),
        )(idx_hbm, out_hbm)

    return run(table, idx_t)


def kernel(inputs, seq_vectors):
    B, S = inputs.shape
    idx_t = inputs.T
    out_t = _lookup_sc(idx_t, seq_vectors)
    return out_t.transpose(2, 1, 0)


# speedup vs baseline: 48.9696x; 1.6674x over previous
import dataclasses
import functools

import jax
import jax.numpy as jnp
from jax.experimental import pallas as pl
from jax.experimental.pallas import tpu as pltpu
from jax.experimental.pallas import tpu_sc as plsc

L = 16
SB = 8
BB = 512


def _lookup_sc(idx_t, table):
    S, B = idx_t.shape
    V, F = table.shape
    mesh = plsc.VectorSubcoreMesh(core_axis_name="c", subcore_axis_name="s")

    cp = pltpu.CompilerParams()
    if "needs_layout_passes" in pltpu.CompilerParams.__dataclass_fields__:
        cp = dataclasses.replace(cp, needs_layout_passes=False)

    @functools.partial(
        pl.kernel,
        out_type=jax.ShapeDtypeStruct((F, S, B), jnp.float32),
        mesh=mesh,
        scratch_types=[pltpu.VMEM((V, F), jnp.float32)],
        compiler_params=cp,
    )
    def run(table_hbm, idx_hbm, out_hbm, table_v):
        pltpu.sync_copy(table_hbm, table_v)

        def body(idx_v, out_v):
            @pl.loop(0, SB)
            def _(s):
                @plsc.parallel_loop(0, BB, step=L, unroll=4)
                def _(b):
                    iv = idx_v[s, pl.ds(b, L)]
                    for f in range(F):
                        vals = plsc.load_gather(
                            table_v, [iv, jnp.full((L,), f, jnp.int32)])
                        out_v[f, s, pl.ds(b, L)] = vals

        pltpu.emit_pipeline(
            body,
            grid=(S // SB, B // BB),
            in_specs=[pl.BlockSpec((SB, BB), lambda i, j: (i, j))],
            out_specs=[pl.BlockSpec((F, SB, BB), lambda i, j: (0, i, j))],
            core_axis_name=("c", "s"),
            dimension_semantics=(pltpu.PARALLEL, pltpu.PARALLEL),
        )(idx_hbm, out_hbm)

    return run(table, idx_t)


def kernel(inputs, seq_vectors):
    B, S = inputs.shape
    idx_t = inputs.T
    out_t = _lookup_sc(idx_t, seq_vectors)
    return out_t.transpose(2, 1, 0)
